# baseline (speedup 1.0000x reference)
import functools

import jax
import jax.numpy as jnp
from jax import lax
from jax.experimental import pallas as pl
from jax.experimental.pallas import tpu as pltpu
from jax.experimental.pallas import tpu_sc as plsc

N0, N1, N2 = 100000, 20000, 2048
N0P = 100352
N1T = 20480
D0, D1 = 16, 16
DIM, OUTC, H = 128, 64, 2
K1SEL = D0 // 2

NC, NS = 2, 16
NW = NC * NS


@functools.lru_cache(maxsize=1)
def _sc_mesh():
    return plsc.VectorSubcoreMesh(core_axis_name="c", subcore_axis_name="s")


def _wid():
    return lax.axis_index("s") * NC + lax.axis_index("c")


def _logits_body(blk, x_ref, wt_ref, b_ref, logit_ref, sig_ref):
    i = pl.program_id(0)
    z = lax.dot_general(wt_ref[...], x_ref[...],
                        (((1,), (1,)), ((), ()))) + b_ref[...]
    logit_ref[:, pl.ds(i * blk, blk)] = z
    sig_ref[:, pl.ds(i * blk, blk)] = jax.nn.sigmoid(z)


def _logits0(x, Wo0, bo0):
    blk = 2048
    grid = (N0P // blk,)
    return pl.pallas_call(
        functools.partial(_logits_body, blk),
        grid=grid,
        in_specs=[
            pl.BlockSpec((blk, DIM), lambda i: (i, 0)),
            pl.BlockSpec((1, DIM), lambda i: (0, 0)),
            pl.BlockSpec((1, 1), lambda i: (0, 0)),
        ],
        out_specs=[
            pl.BlockSpec((1, N0P), lambda i: (0, 0)),
            pl.BlockSpec((1, N0P), lambda i: (0, 0)),
        ],
        out_shape=[
            jax.ShapeDtypeStruct((1, N0P), jnp.float32),
            jax.ShapeDtypeStruct((1, N0P), jnp.float32),
        ],
    )(x, Wo0.reshape(1, DIM), bo0.reshape(1, 1))


def _topk_body(l_hbm, src_hbm, out_hbm, l_v, src_v, sel_v, tmp_v):
    wid = _wid()
    gpw = N1T // NW
    pltpu.sync_copy(l_hbm, l_v)
    pltpu.sync_copy(src_hbm.at[pl.ds(wid * gpw * D0, gpw * D0)], src_v)

    eight = jnp.full((16,), K1SEL, jnp.int32)
    seven = jnp.full((16,), K1SEL - 1, jnp.int32)
    lanes = lax.iota(jnp.int32, 16)

    def body(g, carry):
        idx = src_v[pl.ds(g * D0, 16)]
        row = plsc.load_gather(l_v, [idx])
        d = wid * gpw + g
        col = plsc.load_gather(l_v, [jnp.full((16,), d, jnp.int32)])
        sim = 1.0 - jnp.abs(row - col)
        skeys, _vals = plsc.sort_key_val(sim, lanes, descending=True)
        tmp_v[...] = skeys
        t = plsc.load_gather(tmp_v, [seven])
        gt = sim > t
        n_gt = plsc.all_reduce_population_count(gt)
        eq = sim == t
        csum = lax.cumsum(jnp.where(eq, 1, 0), axis=0)
        sel = jnp.logical_or(gt, jnp.logical_and(eq, csum <= (eight - n_gt)))
        plsc.store_compressed(sel_v.at[pl.ds(g * K1SEL, 16)], idx, mask=sel)
        return carry

    lax.fori_loop(0, gpw, body, 0)
    opw = gpw * K1SEL
    pltpu.sync_copy(sel_v.at[pl.ds(0, opw)], out_hbm.at[pl.ds(wid * opw, opw)])


def _topk_select(l, src0):
    gpw = N1T // NW
    f = pl.kernel(
        _topk_body,
        mesh=_sc_mesh(),
        compiler_params=pltpu.CompilerParams(needs_layout_passes=False),
        out_type=jax.ShapeDtypeStruct((N1T * K1SEL,), jnp.int32),
        scratch_types=[
            pltpu.VMEM((N0P,), jnp.float32),
            pltpu.VMEM((gpw * D0,), jnp.int32),
            pltpu.VMEM((gpw * K1SEL + 16,), jnp.int32),
            pltpu.VMEM((16,), jnp.float32),
        ],
    )
    return f(l, src0)


def _make_gather_t(n_dst_pad, deg, chunk):
    dpw = n_dst_pad // NW
    bpw = dpw * deg
    nch = bpw // chunk
    assert dpw % chunk == 0 and chunk % 8 == 0 and dpw % 16 == 0

    def body(table_hbm, idx_hbm, out_hbm, idx_v, idxt_v, rows_a, rows_b,
             sem_a, sem_b):
        wid = _wid()
        base = wid * bpw
        pltpu.sync_copy(idx_hbm.at[pl.ds(base, bpw)], idx_v)

        lanes = lax.iota(jnp.int32, 16)
        for e in range(deg):
            def tbody(i, carry, e=e):
                srcpos = (i * 16 + lanes) * deg + e
                vals = plsc.load_gather(idx_v, [srcpos])
                idxt_v[pl.ds(e * dpw + i * 16, 16)] = vals
                return carry

            lax.fori_loop(0, dpw // 16, tbody, 0)

        bufs = (rows_a, rows_b)
        sems = (sem_a, sem_b)

        def gather(k):
            pltpu.async_copy(
                table_hbm.at[idxt_v.at[pl.ds(k * chunk, chunk)]],
                bufs[k % 2], sems[k % 2])

        def gwait(k):
            pltpu.make_async_copy(
                table_hbm.at[idxt_v.at[pl.ds(k * chunk, chunk)]],
                bufs[k % 2], sems[k % 2]).wait()

        gather(0)
        for k in range(nch):
            if k + 1 < nch:
                gather(k + 1)
            gwait(k)
            e = (k * chunk) // dpw
            off = (k * chunk) % dpw
            pltpu.sync_copy(
                bufs[k % 2],
                out_hbm.at[e, pl.ds(wid * dpw + off, chunk), :])

    def run(table, idx):
        f = pl.kernel(
            body,
            mesh=_sc_mesh(),
            compiler_params=pltpu.CompilerParams(needs_layout_passes=False),
            out_type=jax.ShapeDtypeStruct((deg, n_dst_pad, DIM), jnp.float32),
            scratch_types=[
                pltpu.VMEM((bpw,), jnp.int32),
                pltpu.VMEM((bpw,), jnp.int32),
                pltpu.VMEM((chunk, DIM), jnp.float32),
                pltpu.VMEM((chunk, DIM), jnp.float32),
                pltpu.SemaphoreType.DMA,
                pltpu.SemaphoreType.DMA,
            ],
        )
        return f(table, idx)

    return run


_gather_l1 = _make_gather_t(N1T, K1SEL, 320)
_gather_l2 = _make_gather_t(N2, D1, 64)


def _attn_body(deg, xd_ref, rows_ref, bsel_ref, csel_ref, wq_ref, wk_ref,
               wv_ref, ws_ref, bq_ref, bv_ref, bs_ref, bng_ref, bnb_ref,
               bnm_ref, bnv_ref, wo_ref, bo_ref, feat_ref, logit_ref):
    xd = xd_ref[...]
    bd = xd.shape[0]
    nsc = 2 * deg
    q = xd @ wq_ref[...] + bq_ref[...]
    rows = rows_ref[...]
    rows_flat = rows.reshape(deg * bd, DIM)
    kk = (rows_flat @ wk_ref[...]).reshape(deg, bd, DIM)
    vv = (rows_flat @ wv_ref[...]).reshape(deg, bd, DIM)
    p3 = kk * q[None]
    bsel = bsel_ref[...]
    csel = csel_ref[...]
    s01 = p3[0] @ bsel[:, :nsc]
    for e in range(1, deg):
        s01 = s01 + p3[e] @ bsel[:, e * nsc:(e + 1) * nsc]
    m0 = s01[:, :deg].max(axis=-1, keepdims=True)
    m1 = s01[:, deg:].max(axis=-1, keepdims=True)
    mC = jnp.concatenate([jnp.broadcast_to(m0, (bd, deg)),
                          jnp.broadcast_to(m1, (bd, deg))], axis=-1)
    ex = jnp.exp(s01 - mC)
    d0 = ex[:, :deg].sum(axis=-1, keepdims=True)
    d1 = ex[:, deg:].sum(axis=-1, keepdims=True)
    dC = jnp.concatenate([jnp.broadcast_to(d0, (bd, deg)),
                          jnp.broadcast_to(d1, (bd, deg))], axis=-1)
    alpha = ex / (dC + 1e-16)
    out = bv_ref[...] + xd @ ws_ref[...] + bs_ref[...]
    for e in range(deg):
        alphaE = alpha @ csel[:, e * DIM:(e + 1) * DIM]
        out = out + alphaE * vv[e]
    rstd = lax.rsqrt(bnv_ref[...] + 1e-5)
    z = (out - bnm_ref[...]) * rstd
    z = z * bng_ref[...] + bnb_ref[...]
    sp = jnp.maximum(z, 0.0) + jnp.log1p(jnp.exp(-jnp.abs(z)))
    feat = z * jnp.tanh(sp)
    feat_ref[...] = feat
    logit_ref[:, pl.ds(pl.program_id(0) * bd, bd)] = lax.dot_general(
        wo_ref[...], feat, (((1,), (1,)), ((), ()))) + bo_ref[...]


@functools.lru_cache(maxsize=4)
def _sel_consts(deg):
    import numpy as np
    nsc = 2 * deg
    inv_sqrt = 1.0 / float(np.sqrt(OUTC))
    B = np.zeros((DIM, deg, nsc), dtype=np.float32)
    C = np.zeros((nsc, deg, DIM), dtype=np.float32)
    for e in range(deg):
        for h in range(H):
            B[h * OUTC:(h + 1) * OUTC, e, h * deg + e] = inv_sqrt
            C[h * deg + e, e, h * OUTC:(h + 1) * OUTC] = 1.0
    return (jnp.asarray(B.reshape(DIM, deg * nsc)),
            jnp.asarray(C.reshape(nsc, deg * DIM)))


def _attn_layer(n_dst, n_dst_pad, deg, blk, x_dst_src, rows_t, Wq, Wk, Wv,
                Ws, bq, bv, bs, bng, bnb, bnm, bnv, Wo, bo):
    nblk = n_dst_pad // blk
    grid = (nblk,)
    nsc = 2 * deg
    bsel, csel = _sel_consts(deg)
    wspec = pl.BlockSpec((DIM, DIM), lambda i: (0, 0))
    bspec = pl.BlockSpec((1, DIM), lambda i: (0, 0))
    return pl.pallas_call(
        functools.partial(_attn_body, deg),
        grid=grid,
        in_specs=[
            pl.BlockSpec((blk, DIM), lambda i: (i, 0)),
            pl.BlockSpec((deg, blk, DIM), lambda i: (0, i, 0)),
            pl.BlockSpec((DIM, deg * nsc), lambda i: (0, 0)),
            pl.BlockSpec((nsc, deg * DIM), lambda i: (0, 0)),
            wspec, wspec, wspec, wspec,
            bspec, bspec, bspec, bspec, bspec, bspec, bspec,
            pl.BlockSpec((1, DIM), lambda i: (0, 0)),
            pl.BlockSpec((1, 1), lambda i: (0, 0)),
        ],
        out_specs=[
            pl.BlockSpec((blk, DIM), lambda i: (i, 0)),
            pl.BlockSpec((1, n_dst_pad), lambda i: (0, 0)),
        ],
        out_shape=[
            jax.ShapeDtypeStruct((n_dst, DIM), jnp.float32),
            jax.ShapeDtypeStruct((1, n_dst_pad), jnp.float32),
        ],
    )(x_dst_src, rows_t, bsel, csel, Wq, Wk, Wv, Ws,
      bq.reshape(1, DIM), bv.reshape(1, DIM), bs.reshape(1, DIM),
      bng.reshape(1, DIM), bnb.reshape(1, DIM), bnm.reshape(1, DIM),
      bnv.reshape(1, DIM), Wo.reshape(1, DIM), bo.reshape(1, 1))


def kernel(x, edge_index_0, edge_index_1, Wq, bq, Wk, bk, Wv, bv, Ws, bs,
           bn_g, bn_b, bn_m, bn_v, Wo, bo):
    src0 = jnp.pad(edge_index_0[0], (0, (N1T - N1) * D0))
    src1 = edge_index_1[0]

    logits0, l = _logits0(x, Wo[0], bo[0])
    t0 = logits0[0, :N2].reshape(N2, 1)

    sel_src = _topk_select(l.reshape(-1), src0)
    rows1 = _gather_l1(x, sel_src)

    feat1, logits1 = _attn_layer(
        N1, N1T, K1SEL, 512, x, rows1, Wq[0], Wk[0], Wv[0], Ws[0],
        bq[0], bv[0], bs[0], bn_g[0], bn_b[0], bn_m[0], bn_v[0], Wo[1], bo[1])
    t1 = logits1[0, :N2].reshape(N2, 1)

    rows2 = _gather_l2(feat1, src1)
    _feat2, logits2 = _attn_layer(
        N2, N2, D1, 256, feat1, rows2, Wq[1], Wk[1], Wv[1], Ws[1],
        bq[1], bv[1], bs[1], bn_g[1], bn_b[1], bn_m[1], bn_v[1], Wo[2], bo[2])
    return (t0, t1, logits2[0].reshape(N2, 1))


# speedup vs baseline: 1.1215x; 1.1215x over previous
import functools

import jax
import jax.numpy as jnp
from jax import lax
from jax.experimental import pallas as pl
from jax.experimental.pallas import tpu as pltpu
from jax.experimental.pallas import tpu_sc as plsc

N0, N1, N2 = 100000, 20000, 2048
N0P = 100352
N1T = 20480
D0, D1 = 16, 16
DIM, OUTC, H = 128, 64, 2
K1SEL = D0 // 2

NC, NS = 2, 16
NW = NC * NS


@functools.lru_cache(maxsize=1)
def _sc_mesh():
    return plsc.VectorSubcoreMesh(core_axis_name="c", subcore_axis_name="s")


def _wid():
    return lax.axis_index("s") * NC + lax.axis_index("c")


def _logits_body(blk, x_ref, wt_ref, b_ref, logit_ref, sig_ref):
    i = pl.program_id(0)
    z = lax.dot_general(wt_ref[...], x_ref[...],
                        (((1,), (1,)), ((), ()))) + b_ref[...]
    logit_ref[:, pl.ds(i * blk, blk)] = z
    sig_ref[:, pl.ds(i * blk, blk)] = jax.nn.sigmoid(z)


def _logits0(x, Wo0, bo0):
    blk = 2048
    grid = (N0P // blk,)
    return pl.pallas_call(
        functools.partial(_logits_body, blk),
        grid=grid,
        in_specs=[
            pl.BlockSpec((blk, DIM), lambda i: (i, 0)),
            pl.BlockSpec((1, DIM), lambda i: (0, 0)),
            pl.BlockSpec((1, 1), lambda i: (0, 0)),
        ],
        out_specs=[
            pl.BlockSpec((1, N0P), lambda i: (0, 0)),
            pl.BlockSpec((1, N0P), lambda i: (0, 0)),
        ],
        out_shape=[
            jax.ShapeDtypeStruct((1, N0P), jnp.float32),
            jax.ShapeDtypeStruct((1, N0P), jnp.float32),
        ],
    )(x, Wo0.reshape(1, DIM), bo0.reshape(1, 1))


def _topk_body(l_hbm, src_hbm, out_hbm, l_v, src_v, sel_v, tmp_v):
    wid = _wid()
    gpw = N1T // NW
    pltpu.sync_copy(l_hbm, l_v)
    pltpu.sync_copy(src_hbm.at[pl.ds(wid * gpw * D0, gpw * D0)], src_v)

    eight = jnp.full((16,), K1SEL, jnp.int32)
    seven = jnp.full((16,), K1SEL - 1, jnp.int32)
    lanes = lax.iota(jnp.int32, 16)

    def body(g, carry):
        idx = src_v[pl.ds(g * D0, 16)]
        row = plsc.load_gather(l_v, [idx])
        d = wid * gpw + g
        col = plsc.load_gather(l_v, [jnp.full((16,), d, jnp.int32)])
        sim = 1.0 - jnp.abs(row - col)
        skeys, _vals = plsc.sort_key_val(sim, lanes, descending=True)
        tmp_v[...] = skeys
        t = plsc.load_gather(tmp_v, [seven])
        gt = sim > t
        n_gt = plsc.all_reduce_population_count(gt)
        eq = sim == t
        csum = lax.cumsum(jnp.where(eq, 1, 0), axis=0)
        sel = jnp.logical_or(gt, jnp.logical_and(eq, csum <= (eight - n_gt)))
        plsc.store_compressed(sel_v.at[pl.ds(g * K1SEL, 16)], idx, mask=sel)
        return carry

    lax.fori_loop(0, gpw, body, 0)
    opw = gpw * K1SEL
    pltpu.sync_copy(sel_v.at[pl.ds(0, opw)], out_hbm.at[pl.ds(wid * opw, opw)])


def _topk_select(l, src0):
    gpw = N1T // NW
    f = pl.kernel(
        _topk_body,
        mesh=_sc_mesh(),
        compiler_params=pltpu.CompilerParams(needs_layout_passes=False),
        out_type=jax.ShapeDtypeStruct((N1T * K1SEL,), jnp.int32),
        scratch_types=[
            pltpu.VMEM((N0P,), jnp.float32),
            pltpu.VMEM((gpw * D0,), jnp.int32),
            pltpu.VMEM((gpw * K1SEL + 16,), jnp.int32),
            pltpu.VMEM((16,), jnp.float32),
        ],
    )
    return f(l, src0)


def _make_gather(n_idx, chunk):
    bpw = n_idx // NW
    nch = bpw // chunk
    assert bpw % chunk == 0 and chunk % 8 == 0

    def body(table_hbm, idx_hbm, out_hbm, idx_v, rows_a, rows_b, sem_a, sem_b):
        wid = _wid()
        base = wid * bpw
        pltpu.sync_copy(idx_hbm.at[pl.ds(base, bpw)], idx_v)

        def gather(c, buf, sem):
            pltpu.async_copy(
                table_hbm.at[idx_v.at[pl.ds(c * chunk, chunk)]], buf, sem)

        def gwait(c, buf, sem):
            pltpu.make_async_copy(
                table_hbm.at[idx_v.at[pl.ds(c * chunk, chunk)]], buf, sem
            ).wait()

        gather(0, rows_a, sem_a)

        def step(c, carry):
            even = (c % 2) == 0

            @pl.when(jnp.logical_and(c + 1 < nch, even))
            def _pf_b():
                gather(c + 1, rows_b, sem_b)

            @pl.when(jnp.logical_and(c + 1 < nch, jnp.logical_not(even)))
            def _pf_a():
                gather(c + 1, rows_a, sem_a)

            @pl.when(even)
            def _drain_a():
                gwait(c, rows_a, sem_a)
                pltpu.sync_copy(
                    rows_a, out_hbm.at[pl.ds(base + c * chunk, chunk), :])

            @pl.when(jnp.logical_not(even))
            def _drain_b():
                gwait(c, rows_b, sem_b)
                pltpu.sync_copy(
                    rows_b, out_hbm.at[pl.ds(base + c * chunk, chunk), :])
            return carry

        lax.fori_loop(0, nch, step, 0)

    def run(table, idx):
        f = pl.kernel(
            body,
            mesh=_sc_mesh(),
            compiler_params=pltpu.CompilerParams(needs_layout_passes=False),
            out_type=jax.ShapeDtypeStruct((n_idx, DIM), jnp.float32),
            scratch_types=[
                pltpu.VMEM((bpw,), jnp.int32),
                pltpu.VMEM((chunk, DIM), jnp.float32),
                pltpu.VMEM((chunk, DIM), jnp.float32),
                pltpu.SemaphoreType.DMA,
                pltpu.SemaphoreType.DMA,
            ],
        )
        return f(table, idx)

    return run


_gather_l1 = _make_gather(N1T * K1SEL, 320)
_gather_l2 = _make_gather(N2 * D1, 256)


def _attn_body(deg, xd_ref, rows_ref, bsel_ref, csel_ref, wq_ref, wk_ref,
               wv_ref, ws_ref, bq_ref, bv_ref, bs_ref, bng_ref, bnb_ref,
               bnm_ref, bnv_ref, wo_ref, bo_ref, feat_ref, logit_ref):
    xd = xd_ref[...]
    bd = xd.shape[0]
    nsc = 2 * deg
    q = xd @ wq_ref[...] + bq_ref[...]
    rows = rows_ref[...]
    rows_flat = rows.reshape(deg * bd, DIM)
    kk = (rows_flat @ wk_ref[...]).reshape(deg, bd, DIM)
    vv = (rows_flat @ wv_ref[...]).reshape(deg, bd, DIM)
    p3 = kk * q[None]
    bsel = bsel_ref[...]
    csel = csel_ref[...]
    s01 = p3[0] @ bsel[:, :nsc]
    for e in range(1, deg):
        s01 = s01 + p3[e] @ bsel[:, e * nsc:(e + 1) * nsc]
    m0 = s01[:, :deg].max(axis=-1, keepdims=True)
    m1 = s01[:, deg:].max(axis=-1, keepdims=True)
    mC = jnp.concatenate([jnp.broadcast_to(m0, (bd, deg)),
                          jnp.broadcast_to(m1, (bd, deg))], axis=-1)
    ex = jnp.exp(s01 - mC)
    d0 = ex[:, :deg].sum(axis=-1, keepdims=True)
    d1 = ex[:, deg:].sum(axis=-1, keepdims=True)
    dC = jnp.concatenate([jnp.broadcast_to(d0, (bd, deg)),
                          jnp.broadcast_to(d1, (bd, deg))], axis=-1)
    alpha = ex / (dC + 1e-16)
    out = bv_ref[...] + xd @ ws_ref[...] + bs_ref[...]
    for e in range(deg):
        alphaE = alpha @ csel[:, e * DIM:(e + 1) * DIM]
        out = out + alphaE * vv[e]
    rstd = lax.rsqrt(bnv_ref[...] + 1e-5)
    z = (out - bnm_ref[...]) * rstd
    z = z * bng_ref[...] + bnb_ref[...]
    sp = jnp.maximum(z, 0.0) + jnp.log1p(jnp.exp(-jnp.abs(z)))
    feat = z * jnp.tanh(sp)
    feat_ref[...] = feat
    logit_ref[:, pl.ds(pl.program_id(0) * bd, bd)] = lax.dot_general(
        wo_ref[...], feat, (((1,), (1,)), ((), ()))) + bo_ref[...]


@functools.lru_cache(maxsize=4)
def _sel_consts(deg):
    import numpy as np
    nsc = 2 * deg
    inv_sqrt = 1.0 / float(np.sqrt(OUTC))
    B = np.zeros((DIM, deg, nsc), dtype=np.float32)
    C = np.zeros((nsc, deg, DIM), dtype=np.float32)
    for e in range(deg):
        for h in range(H):
            B[h * OUTC:(h + 1) * OUTC, e, h * deg + e] = inv_sqrt
            C[h * deg + e, e, h * OUTC:(h + 1) * OUTC] = 1.0
    return (jnp.asarray(B.reshape(DIM, deg * nsc)),
            jnp.asarray(C.reshape(nsc, deg * DIM)))


def _attn_layer(n_dst, n_dst_pad, deg, blk, x_dst_src, rows_t, Wq, Wk, Wv,
                Ws, bq, bv, bs, bng, bnb, bnm, bnv, Wo, bo):
    nblk = n_dst_pad // blk
    grid = (nblk,)
    nsc = 2 * deg
    bsel, csel = _sel_consts(deg)
    wspec = pl.BlockSpec((DIM, DIM), lambda i: (0, 0))
    bspec = pl.BlockSpec((1, DIM), lambda i: (0, 0))
    return pl.pallas_call(
        functools.partial(_attn_body, deg),
        grid=grid,
        in_specs=[
            pl.BlockSpec((blk, DIM), lambda i: (i, 0)),
            pl.BlockSpec((deg, blk, DIM), lambda i: (0, i, 0)),
            pl.BlockSpec((DIM, deg * nsc), lambda i: (0, 0)),
            pl.BlockSpec((nsc, deg * DIM), lambda i: (0, 0)),
            wspec, wspec, wspec, wspec,
            bspec, bspec, bspec, bspec, bspec, bspec, bspec,
            pl.BlockSpec((1, DIM), lambda i: (0, 0)),
            pl.BlockSpec((1, 1), lambda i: (0, 0)),
        ],
        out_specs=[
            pl.BlockSpec((blk, DIM), lambda i: (i, 0)),
            pl.BlockSpec((1, n_dst_pad), lambda i: (0, 0)),
        ],
        out_shape=[
            jax.ShapeDtypeStruct((n_dst, DIM), jnp.float32),
            jax.ShapeDtypeStruct((1, n_dst_pad), jnp.float32),
        ],
    )(x_dst_src, rows_t, bsel, csel, Wq, Wk, Wv, Ws,
      bq.reshape(1, DIM), bv.reshape(1, DIM), bs.reshape(1, DIM),
      bng.reshape(1, DIM), bnb.reshape(1, DIM), bnm.reshape(1, DIM),
      bnv.reshape(1, DIM), Wo.reshape(1, DIM), bo.reshape(1, 1))


def kernel(x, edge_index_0, edge_index_1, Wq, bq, Wk, bk, Wv, bv, Ws, bs,
           bn_g, bn_b, bn_m, bn_v, Wo, bo):
    src0 = jnp.pad(edge_index_0[0], (0, (N1T - N1) * D0))
    src1_t = edge_index_1[0].reshape(N2, D1).T.reshape(-1)

    logits0, l = _logits0(x, Wo[0], bo[0])
    t0 = logits0[0, :N2].reshape(N2, 1)

    sel_src = _topk_select(l.reshape(-1), src0)
    sel_t = sel_src.reshape(N1T, K1SEL).T.reshape(-1)
    rows1 = _gather_l1(x, sel_t).reshape(K1SEL, N1T, DIM)

    feat1, logits1 = _attn_layer(
        N1, N1T, K1SEL, 512, x, rows1, Wq[0], Wk[0], Wv[0], Ws[0],
        bq[0], bv[0], bs[0], bn_g[0], bn_b[0], bn_m[0], bn_v[0], Wo[1], bo[1])
    t1 = logits1[0, :N2].reshape(N2, 1)

    rows2 = _gather_l2(feat1, src1_t).reshape(D1, N2, DIM)
    _feat2, logits2 = _attn_layer(
        N2, N2, D1, 256, feat1, rows2, Wq[1], Wk[1], Wv[1], Ws[1],
        bq[1], bv[1], bs[1], bn_g[1], bn_b[1], bn_m[1], bn_v[1], Wo[2], bo[2])
    return (t0, t1, logits2[0].reshape(N2, 1))


# speedup vs baseline: 1.6323x; 1.4555x over previous
import functools

import jax
import jax.numpy as jnp
from jax import lax
from jax.experimental import pallas as pl
from jax.experimental.pallas import tpu as pltpu
from jax.experimental.pallas import tpu_sc as plsc

N0, N1, N2 = 100000, 20000, 2048
N0P = 100352
N1T = 20480
D0, D1 = 16, 16
DIM, OUTC, H = 128, 64, 2
K1SEL = D0 // 2

NC, NS = 2, 16
NW = NC * NS


@functools.lru_cache(maxsize=1)
def _sc_mesh():
    return plsc.VectorSubcoreMesh(core_axis_name="c", subcore_axis_name="s")


def _wid():
    return lax.axis_index("s") * NC + lax.axis_index("c")


def _logits_body(blk, x_ref, wt_ref, b_ref, logit_ref, sig_ref):
    i = pl.program_id(0)
    z = lax.dot_general(wt_ref[...], x_ref[...],
                        (((1,), (1,)), ((), ()))) + b_ref[...]
    logit_ref[:, pl.ds(i * blk, blk)] = z
    sig_ref[:, pl.ds(i * blk, blk)] = jax.nn.sigmoid(z)


def _logits0(x, Wo0, bo0):
    blk = 2048
    grid = (N0P // blk,)
    return pl.pallas_call(
        functools.partial(_logits_body, blk),
        grid=grid,
        in_specs=[
            pl.BlockSpec((blk, DIM), lambda i: (i, 0)),
            pl.BlockSpec((1, DIM), lambda i: (0, 0)),
            pl.BlockSpec((1, 1), lambda i: (0, 0)),
        ],
        out_specs=[
            pl.BlockSpec((1, N0P), lambda i: (0, 0)),
            pl.BlockSpec((1, N0P), lambda i: (0, 0)),
        ],
        out_shape=[
            jax.ShapeDtypeStruct((1, N0P), jnp.float32),
            jax.ShapeDtypeStruct((1, N0P), jnp.float32),
        ],
    )(x, Wo0.reshape(1, DIM), bo0.reshape(1, 1))


def _topk_body(l_hbm, src_hbm, out_hbm, l_v, src_v, sel_v, tmp_v):
    wid = _wid()
    gpw = N1T // NW
    pltpu.sync_copy(l_hbm, l_v)
    pltpu.sync_copy(src_hbm.at[pl.ds(wid * gpw * D0, gpw * D0)], src_v)

    eight = jnp.full((16,), K1SEL, jnp.int32)
    seven = jnp.full((16,), K1SEL - 1, jnp.int32)
    lanes = lax.iota(jnp.int32, 16)

    def body(g, carry):
        idx = src_v[pl.ds(g * D0, 16)]
        row = plsc.load_gather(l_v, [idx])
        d = wid * gpw + g
        col = plsc.load_gather(l_v, [jnp.full((16,), d, jnp.int32)])
        sim = 1.0 - jnp.abs(row - col)
        skeys, _vals = plsc.sort_key_val(sim, lanes, descending=True)
        tmp_v[...] = skeys
        t = plsc.load_gather(tmp_v, [seven])
        gt = sim > t
        n_gt = plsc.all_reduce_population_count(gt)
        eq = sim == t
        csum = lax.cumsum(jnp.where(eq, 1, 0), axis=0)
        sel = jnp.logical_or(gt, jnp.logical_and(eq, csum <= (eight - n_gt)))
        plsc.store_compressed(sel_v.at[pl.ds(g * K1SEL, 16)], idx, mask=sel)
        return carry

    lax.fori_loop(0, gpw, body, 0)
    opw = gpw * K1SEL
    pltpu.sync_copy(sel_v.at[pl.ds(0, opw)], out_hbm.at[pl.ds(wid * opw, opw)])


def _topk_select(l, src0):
    gpw = N1T // NW
    f = pl.kernel(
        _topk_body,
        mesh=_sc_mesh(),
        compiler_params=pltpu.CompilerParams(needs_layout_passes=False),
        out_type=jax.ShapeDtypeStruct((N1T * K1SEL,), jnp.int32),
        scratch_types=[
            pltpu.VMEM((N0P,), jnp.float32),
            pltpu.VMEM((gpw * D0,), jnp.int32),
            pltpu.VMEM((gpw * K1SEL + 16,), jnp.int32),
            pltpu.VMEM((16,), jnp.float32),
        ],
    )
    return f(l, src0)


def _make_gather(n_idx, chunk):
    bpw = n_idx // NW
    nch = bpw // chunk
    assert bpw % chunk == 0 and chunk % 8 == 0

    def body(table_hbm, idx_hbm, out_hbm, idx_v, rows_a, rows_b, sem_a, sem_b):
        wid = _wid()
        base = wid * bpw
        pltpu.sync_copy(idx_hbm.at[pl.ds(base, bpw)], idx_v)

        def gather(c, buf, sem):
            pltpu.async_copy(
                table_hbm.at[idx_v.at[pl.ds(c * chunk, chunk)]], buf, sem)

        def gwait(c, buf, sem):
            pltpu.make_async_copy(
                table_hbm.at[idx_v.at[pl.ds(c * chunk, chunk)]], buf, sem
            ).wait()

        gather(0, rows_a, sem_a)

        def step(c, carry):
            even = (c % 2) == 0

            @pl.when(jnp.logical_and(c + 1 < nch, even))
            def _pf_b():
                gather(c + 1, rows_b, sem_b)

            @pl.when(jnp.logical_and(c + 1 < nch, jnp.logical_not(even)))
            def _pf_a():
                gather(c + 1, rows_a, sem_a)

            @pl.when(even)
            def _drain_a():
                gwait(c, rows_a, sem_a)
                pltpu.sync_copy(
                    rows_a, out_hbm.at[pl.ds(base + c * chunk, chunk), :])

            @pl.when(jnp.logical_not(even))
            def _drain_b():
                gwait(c, rows_b, sem_b)
                pltpu.sync_copy(
                    rows_b, out_hbm.at[pl.ds(base + c * chunk, chunk), :])
            return carry

        lax.fori_loop(0, nch, step, 0)

    def run(table, idx):
        f = pl.kernel(
            body,
            mesh=_sc_mesh(),
            compiler_params=pltpu.CompilerParams(needs_layout_passes=False),
            out_type=jax.ShapeDtypeStruct((n_idx, DIM), jnp.float32),
            scratch_types=[
                pltpu.VMEM((bpw,), jnp.int32),
                pltpu.VMEM((chunk, DIM), jnp.float32),
                pltpu.VMEM((chunk, DIM), jnp.float32),
                pltpu.SemaphoreType.DMA,
                pltpu.SemaphoreType.DMA,
            ],
        )
        return f(table, idx)

    return run


_gather_l1 = _make_gather(N1T * K1SEL, 320)
_gather_l2 = _make_gather(N2 * D1, 256)


def _attn_body(deg, xd_ref, rows_ref, bsel_ref, csel_ref, wq_ref, wk_ref,
               wv_ref, ws_ref, bq_ref, bv_ref, bs_ref, bng_ref, bnb_ref,
               bnm_ref, bnv_ref, wo_ref, bo_ref, feat_ref, logit_ref):
    xd = xd_ref[...]
    bd = xd.shape[0]
    nsc = 2 * deg
    q = xd @ wq_ref[...] + bq_ref[...]
    rows = rows_ref[...]
    rows_flat = rows.reshape(deg * bd, DIM)
    kk = (rows_flat @ wk_ref[...]).reshape(deg, bd, DIM)
    vv = (rows_flat @ wv_ref[...]).reshape(deg, bd, DIM)
    p3 = kk * q[None]
    bsel = bsel_ref[...]
    csel = csel_ref[...]
    s01 = p3[0] @ bsel[:, :nsc]
    for e in range(1, deg):
        s01 = s01 + p3[e] @ bsel[:, e * nsc:(e + 1) * nsc]
    m0 = s01[:, :deg].max(axis=-1, keepdims=True)
    m1 = s01[:, deg:].max(axis=-1, keepdims=True)
    mC = jnp.concatenate([jnp.broadcast_to(m0, (bd, deg)),
                          jnp.broadcast_to(m1, (bd, deg))], axis=-1)
    ex = jnp.exp(s01 - mC)
    d0 = ex[:, :deg].sum(axis=-1, keepdims=True)
    d1 = ex[:, deg:].sum(axis=-1, keepdims=True)
    dC = jnp.concatenate([jnp.broadcast_to(d0, (bd, deg)),
                          jnp.broadcast_to(d1, (bd, deg))], axis=-1)
    alpha = ex / (dC + 1e-16)
    out = bv_ref[...] + xd @ ws_ref[...] + bs_ref[...]
    for e in range(deg):
        alphaE = alpha @ csel[:, e * DIM:(e + 1) * DIM]
        out = out + alphaE * vv[e]
    rstd = lax.rsqrt(bnv_ref[...] + 1e-5)
    z = (out - bnm_ref[...]) * rstd
    z = z * bng_ref[...] + bnb_ref[...]
    sp = jnp.maximum(z, 0.0) + jnp.log1p(jnp.exp(-jnp.abs(z)))
    feat = z * jnp.tanh(sp)
    feat_ref[...] = feat
    logit_ref[:, pl.ds(pl.program_id(0) * bd, bd)] = lax.dot_general(
        wo_ref[...], feat, (((1,), (1,)), ((), ()))) + bo_ref[...]


@functools.lru_cache(maxsize=4)
def _sel_consts(deg):
    import numpy as np
    nsc = 2 * deg
    inv_sqrt = 1.0 / float(np.sqrt(OUTC))
    B = np.zeros((DIM, deg, nsc), dtype=np.float32)
    C = np.zeros((nsc, deg, DIM), dtype=np.float32)
    for e in range(deg):
        for h in range(H):
            B[h * OUTC:(h + 1) * OUTC, e, h * deg + e] = inv_sqrt
            C[h * deg + e, e, h * OUTC:(h + 1) * OUTC] = 1.0
    return (jnp.asarray(B.reshape(DIM, deg * nsc)),
            jnp.asarray(C.reshape(nsc, deg * DIM)))


def _attn_layer(n_dst, n_dst_pad, deg, blk, x_dst_src, rows_t, Wq, Wk, Wv,
                Ws, bq, bv, bs, bng, bnb, bnm, bnv, Wo, bo):
    nblk = n_dst_pad // blk
    grid = (nblk,)
    nsc = 2 * deg
    bsel, csel = _sel_consts(deg)
    wspec = pl.BlockSpec((DIM, DIM), lambda i: (0, 0))
    bspec = pl.BlockSpec((1, DIM), lambda i: (0, 0))
    return pl.pallas_call(
        functools.partial(_attn_body, deg),
        grid=grid,
        in_specs=[
            pl.BlockSpec((blk, DIM), lambda i: (i, 0)),
            pl.BlockSpec((deg, blk, DIM), lambda i: (0, i, 0)),
            pl.BlockSpec((DIM, deg * nsc), lambda i: (0, 0)),
            pl.BlockSpec((nsc, deg * DIM), lambda i: (0, 0)),
            wspec, wspec, wspec, wspec,
            bspec, bspec, bspec, bspec, bspec, bspec, bspec,
            pl.BlockSpec((1, DIM), lambda i: (0, 0)),
            pl.BlockSpec((1, 1), lambda i: (0, 0)),
        ],
        out_specs=[
            pl.BlockSpec((blk, DIM), lambda i: (i, 0)),
            pl.BlockSpec((1, n_dst_pad), lambda i: (0, 0)),
        ],
        out_shape=[
            jax.ShapeDtypeStruct((n_dst, DIM), jnp.float32),
            jax.ShapeDtypeStruct((1, n_dst_pad), jnp.float32),
        ],
    )(x_dst_src, rows_t, bsel, csel, Wq, Wk, Wv, Ws,
      bq.reshape(1, DIM), bv.reshape(1, DIM), bs.reshape(1, DIM),
      bng.reshape(1, DIM), bnb.reshape(1, DIM), bnm.reshape(1, DIM),
      bnv.reshape(1, DIM), Wo.reshape(1, DIM), bo.reshape(1, 1))


def kernel(x, edge_index_0, edge_index_1, Wq, bq, Wk, bk, Wv, bv, Ws, bs,
           bn_g, bn_b, bn_m, bn_v, Wo, bo):
    pad_idx = jnp.arange(N1 * D0, N1T * D0, dtype=jnp.int32) % N0
    src0 = jnp.concatenate([edge_index_0[0], pad_idx])
    src1_t = edge_index_1[0].reshape(N2, D1).T.reshape(-1)

    logits0, l = _logits0(x, Wo[0], bo[0])
    t0 = logits0[0, :N2].reshape(N2, 1)

    sel_src = _topk_select(l.reshape(-1), src0)
    sel_t = sel_src.reshape(N1T, K1SEL).T.reshape(-1)
    rows1 = _gather_l1(x, sel_t).reshape(K1SEL, N1T, DIM)

    feat1, logits1 = _attn_layer(
        N1, N1T, K1SEL, 512, x, rows1, Wq[0], Wk[0], Wv[0], Ws[0],
        bq[0], bv[0], bs[0], bn_g[0], bn_b[0], bn_m[0], bn_v[0], Wo[1], bo[1])
    t1 = logits1[0, :N2].reshape(N2, 1)

    rows2 = _gather_l2(feat1, src1_t).reshape(D1, N2, DIM)
    _feat2, logits2 = _attn_layer(
        N2, N2, D1, 256, feat1, rows2, Wq[1], Wk[1], Wv[1], Ws[1],
        bq[1], bv[1], bs[1], bn_g[1], bn_b[1], bn_m[1], bn_v[1], Wo[2], bo[2])
    return (t0, t1, logits2[0].reshape(N2, 1))
